# baseline (speedup 1.0000x reference)
import functools

import jax
import jax.numpy as jnp
from jax import lax
from jax.experimental import pallas as pl
from jax.experimental.pallas import tpu as pltpu
from jax.experimental.pallas import tpu_sc as plsc

NC = 2
NS = 16
L = 128
IB = 8
S = 4


def _sc_mesh():
    return plsc.VectorSubcoreMesh(
        core_axis_name="c", subcore_axis_name="s", num_cores=NC,
        num_subcores=NS)


def _zero_zbuf(zbuf, zb_rows, width):
    z16 = jnp.zeros((16,), jnp.float32)

    def zrow(i, _):
        for w in range(width // 16):
            zbuf[i, pl.ds(w * 16, 16)] = z16
        return 0

    lax.fori_loop(0, zb_rows, zrow, 0)


def _zero_acc_slice(acc, zbuf, tid, npt, zb):
    def body(z, _):
        pltpu.sync_copy(zbuf, acc.at[pl.ds(tid * npt + z * zb, zb)])
        return 0

    lax.fori_loop(0, npt // zb, body, 0)


def _dump_acc_slice(acc, out_hbm, tid, npt, zb):
    def body(z, _):
        r0 = tid * npt + z * zb
        pltpu.sync_copy(acc.at[pl.ds(r0, zb)], out_hbm.at[pl.ds(r0, zb)])
        return 0

    lax.fori_loop(0, npt // zb, body, 0)


def _pipe_block(fire_fetch, sblk, rows_list, acc, gsems, ssems):
    prev = [None] * S
    for p in range(IB // S):
        j0 = S * p
        g = []
        for j in range(S):
            if prev[j] is not None:
                prev[j].wait()
            g.append(fire_fetch(j0 + j, rows_list[j], gsems[j]))
        for j in range(S):
            g[j].wait()
            prev[j] = pltpu.async_copy(rows_list[j],
                                       acc.at[sblk.at[j0 + j]],
                                       ssems[j], add=True)
    for d in prev:
        d.wait()


def _make_gsum_kernel(n_out, h, e_pad, npt, zb, kpt):
    kb = e_pad // L
    assert kb == NS * kpt and kpt % IB == 0
    nblk = kpt // IB

    out4 = tuple(jax.ShapeDtypeStruct((n_out, h), jnp.float32)
                 for _ in range(4))
    scratch = (
        [pltpu.VMEM((IB, L), jnp.int32)] * 2
        + [pltpu.VMEM((L, h), jnp.float32)] * S
        + [pltpu.VMEM((zb, h), jnp.float32)]
        + [pltpu.VMEM_SHARED((n_out, h), jnp.float32)]
        + [pltpu.SemaphoreType.DMA] * (2 * S))

    @functools.partial(pl.kernel, out_type=out4, mesh=_sc_mesh(),
                       scratch_types=scratch,
                       compiler_params=pltpu.CompilerParams(
                           use_tc_tiling_on_sc=False))
    def gsum(h_lo, h_hi, gsrc2, gdst2, ssrc2, sdst2,
             gin_lo, gin_hi, gout_lo, gout_hi,
             gblk, sblk, *rest):
        rows_list = list(rest[:S])
        zbuf = rest[S]
        acc = rest[S + 1]
        gsems = list(rest[S + 2:2 * S + 2])
        ssems = list(rest[2 * S + 2:])
        cid = lax.axis_index("c")
        tid = lax.axis_index("s")

        _zero_zbuf(zbuf, zb, h)

        def run_pass(gidx2, sidx2, table):
            def blk(b, _):
                b0 = tid * kpt + b * IB
                pltpu.sync_copy(gidx2.at[pl.ds(b0, IB)], gblk)
                pltpu.sync_copy(sidx2.at[pl.ds(b0, IB)], sblk)
                _pipe_block(
                    lambda j, dst, sem: pltpu.async_copy(
                        table.at[gblk.at[j]], dst, sem),
                    sblk, rows_list, acc, gsems, ssems)
                return 0

            lax.fori_loop(0, nblk, blk, 0)

        def flow(table, o_in, o_out):
            _zero_acc_slice(acc, zbuf, tid, npt, zb)
            plsc.subcore_barrier()
            run_pass(gsrc2, sdst2, table)
            plsc.subcore_barrier()
            _dump_acc_slice(acc, o_in, tid, npt, zb)
            _zero_acc_slice(acc, zbuf, tid, npt, zb)
            plsc.subcore_barrier()
            run_pass(gdst2, ssrc2, table)
            plsc.subcore_barrier()
            _dump_acc_slice(acc, o_out, tid, npt, zb)

        @pl.when(cid == 0)
        def _():
            flow(h_lo, gin_lo, gout_lo)

        @pl.when(cid == 1)
        def _():
            flow(h_hi, gin_hi, gout_hi)

    return gsum


def _make_attr_kernel(n_out, e_pad, npt, zb, kpt):
    w = 16
    nblk = kpt // IB

    out2 = (jax.ShapeDtypeStruct((n_out, w), jnp.float32),
            jax.ShapeDtypeStruct((n_out, w), jnp.float32))
    scratch = (
        [pltpu.VMEM((IB, L), jnp.int32)]
        + [pltpu.VMEM((L, w), jnp.float32)] * S
        + [pltpu.VMEM((zb, w), jnp.float32)]
        + [pltpu.VMEM_SHARED((n_out, w), jnp.float32)]
        + [pltpu.SemaphoreType.DMA] * (2 * S))

    @functools.partial(pl.kernel, out_type=out2, mesh=_sc_mesh(),
                       scratch_types=scratch,
                       compiler_params=pltpu.CompilerParams(
                           use_tc_tiling_on_sc=False))
    def attr_sum(attr16, sdst2, ssrc2, a_in, a_out, sblk, *rest):
        rows_list = list(rest[:S])
        zbuf = rest[S]
        acc = rest[S + 1]
        gsems = list(rest[S + 2:2 * S + 2])
        ssems = list(rest[2 * S + 2:])
        cid = lax.axis_index("c")
        tid = lax.axis_index("s")

        _zero_zbuf(zbuf, zb, w)

        def flow(sidx2, out_hbm):
            _zero_acc_slice(acc, zbuf, tid, npt, zb)
            plsc.subcore_barrier()

            def blk(b, _):
                b0 = tid * kpt + b * IB
                pltpu.sync_copy(sidx2.at[pl.ds(b0, IB)], sblk)
                _pipe_block(
                    lambda j, dst, sem: pltpu.async_copy(
                        attr16.at[pl.ds((b0 + j) * L, L)], dst, sem),
                    sblk, rows_list, acc, gsems, ssems)
                return 0

            lax.fori_loop(0, nblk, blk, 0)
            plsc.subcore_barrier()
            _dump_acc_slice(acc, out_hbm, tid, npt, zb)

        @pl.when(cid == 0)
        def _():
            flow(sdst2, a_in)

        @pl.when(cid == 1)
        def _():
            flow(ssrc2, a_out)

    return attr_sum


def _round_body(lo, hi, ginlo, ginhi, goutlo, gouthi, ain, aout, indeg,
                outdeg, wefs, wefd, wers, werd, wfa, wra,
                wihr, wihz, wihn, whhr, whhz, whhn,
                bir, biz, bin_, bhr, bhz, bhn, olo, ohi):
    h = jnp.concatenate([lo[...], hi[...]], axis=1)
    gin = jnp.concatenate([ginlo[...], ginhi[...]], axis=1)
    gout = jnp.concatenate([goutlo[...], gouthi[...]], axis=1)
    mm = lambda a, b: jnp.dot(a, b, preferred_element_type=jnp.float32)
    agg = (mm(gin, wefs[...]) + mm(gout, werd[...])
           + indeg[...] * mm(h, wefd[...]) + outdeg[...] * mm(h, wers[...])
           + mm(ain[...], wfa[...]) + mm(aout[...], wra[...]))
    r = jax.nn.sigmoid(mm(agg, wihr[...]) + bir[...]
                       + mm(h, whhr[...]) + bhr[...])
    z = jax.nn.sigmoid(mm(agg, wihz[...]) + biz[...]
                       + mm(h, whhz[...]) + bhz[...])
    nn = jnp.tanh(mm(agg, wihn[...]) + bin_[...]
                  + r * (mm(h, whhn[...]) + bhn[...]))
    hn = (1.0 - z) * nn + z * h
    half = hn.shape[1] // 2
    olo[...] = hn[:, :half]
    ohi[...] = hn[:, half:]


def _round_agg_body(lo, hi, ginlo, ginhi, goutlo, gouthi, ain, aout, indeg,
                    outdeg, wefs, wefd, wers, werd, wfa, wra,
                    wihr, wihz, wihn, whhr, whhz, whhn,
                    bir, biz, bin_, bhr, bhz, bhn,
                    wfm, bfm, wgm, bgm, olo, ohi, oagg):
    _round_body(lo, hi, ginlo, ginhi, goutlo, gouthi, ain, aout, indeg,
                outdeg, wefs, wefd, wers, werd, wfa, wra,
                wihr, wihz, wihn, whhr, whhz, whhn,
                bir, biz, bin_, bhr, bhz, bhn, olo, ohi)
    hn = jnp.concatenate([olo[...], ohi[...]], axis=1)
    mm = lambda a, b: jnp.dot(a, b, preferred_element_type=jnp.float32)
    f = mm(hn, wfm[...]) + bfm[...]
    g = jax.nn.sigmoid(mm(hn, wgm[...]) + bgm[...])
    p = jnp.sum(f * g, axis=0, keepdims=True)

    @pl.when(pl.program_id(0) == 0)
    def _():
        oagg[...] = jnp.zeros_like(oagg)

    oagg[...] += p


def _split_body(x, olo, ohi):
    half = x.shape[1] // 2
    olo[...] = x[:, :half]
    ohi[...] = x[:, half:]


def kernel(x, edge_index, edge_attr, Wef, bef, Wer, ber, W_ih, W_hh,
           b_ih, b_hh, Wfm, bfm, Wgm, bgm):
    n, d = x.shape
    h = d // 2
    e = edge_index.shape[1]
    es = edge_attr.shape[1]
    rounds = Wef.shape[0]
    hg = Wfm.shape[1]

    kpt = IB * -(-e // (NS * IB * L))
    e_pad = NS * kpt * L
    n_out = -(-n // (NS * 8)) * (NS * 8)
    npt = n_out // NS
    zb = next(c for c in (512, 256, 184, 136, 128, 64, 32, 16, 8)
              if npt % c == 0)
    trash = n_out - 8

    src = edge_index[0]
    dst = edge_index[1]
    pad_g = jnp.zeros((e_pad - e,), jnp.int32)
    pad_s = jnp.full((e_pad - e,), trash, jnp.int32)
    gsrc2 = jnp.concatenate([src, pad_g]).reshape(-1, L)
    gdst2 = jnp.concatenate([dst, pad_g]).reshape(-1, L)
    ssrc2 = jnp.concatenate([src, pad_s]).reshape(-1, L)
    sdst2 = jnp.concatenate([dst, pad_s]).reshape(-1, L)
    be = e_pad // 128
    nblk_a = e_pad // be
    nreal = -(-e // be)

    def _attr16_body(attr, out):
        i = pl.program_id(0)
        ridx = i * be + lax.broadcasted_iota(jnp.int32, (be, 1), 0)
        mask = (ridx < e).astype(jnp.float32)
        out[...] = jnp.concatenate(
            [attr[...] * mask, mask,
             jnp.zeros((be, 16 - es - 1), jnp.float32)], axis=1)

    attr16 = pl.pallas_call(
        _attr16_body,
        grid=(nblk_a,),
        in_specs=[pl.BlockSpec((be, es),
                               lambda i: (jnp.minimum(i, nreal - 1), 0))],
        out_specs=pl.BlockSpec((be, 16), lambda i: (i, 0)),
        out_shape=jax.ShapeDtypeStruct((e_pad, 16), jnp.float32),
    )(edge_attr)

    attr_kernel = _make_attr_kernel(n_out, e_pad, npt, zb, kpt)
    a_in, a_out = attr_kernel(attr16, sdst2, ssrc2)
    indeg = a_in[:, es:es + 1]
    outdeg = a_out[:, es:es + 1]

    gsum = _make_gsum_kernel(n_out, h, e_pad, npt, zb, kpt)

    bn = 2000
    grid = (n // bn,)
    row_spec = lambda w: pl.BlockSpec((bn, w), lambda i: (i, 0))
    full_spec = lambda a, b: pl.BlockSpec((a, b), lambda i: (0, 0))

    wspecs = ([full_spec(d, 2 * d)] * 2 + [full_spec(d, 2 * d)] * 2
              + [full_spec(16, 2 * d)] * 2
              + [full_spec(2 * d, d)] * 3 + [full_spec(d, d)] * 3
              + [full_spec(1, d)] * 6)
    round_call = pl.pallas_call(
        _round_body,
        grid=grid,
        in_specs=[row_spec(h)] * 6 + [row_spec(16)] * 2 + [row_spec(1)] * 2
                 + wspecs,
        out_specs=[row_spec(h), row_spec(h)],
        out_shape=[jax.ShapeDtypeStruct((n, h), jnp.float32)] * 2,
    )

    last_call = pl.pallas_call(
        _round_agg_body,
        grid=grid,
        in_specs=[row_spec(h)] * 6 + [row_spec(16)] * 2 + [row_spec(1)] * 2
                 + wspecs + [full_spec(d, hg), full_spec(1, hg),
                             full_spec(d, hg), full_spec(1, hg)],
        out_specs=[row_spec(h), row_spec(h),
                   pl.BlockSpec((1, hg), lambda i: (0, 0))],
        out_shape=[jax.ShapeDtypeStruct((n, h), jnp.float32)] * 2
                  + [jax.ShapeDtypeStruct((1, hg), jnp.float32)],
    )

    h_lo, h_hi = pl.pallas_call(
        _split_body,
        grid=grid,
        in_specs=[pl.BlockSpec((bn, d), lambda i: (i, 0))],
        out_specs=[row_spec(h), row_spec(h)],
        out_shape=[jax.ShapeDtypeStruct((n, h), jnp.float32)] * 2,
    )(x)

    h_g = None
    for i in range(rounds):
        gin_lo, gin_hi, gout_lo, gout_hi = gsum(
            h_lo, h_hi, gsrc2, gdst2, ssrc2, sdst2)
        wef, wer = Wef[i], Wer[i]
        wfa = jnp.concatenate(
            [wef[2 * d:], bef[i][None, :],
             jnp.zeros((16 - es - 1, 2 * d), jnp.float32)], axis=0)
        wra = jnp.concatenate(
            [wer[2 * d:], ber[i][None, :],
             jnp.zeros((16 - es - 1, 2 * d), jnp.float32)], axis=0)
        wih, whh = W_ih[i], W_hh[i]
        args = (
            h_lo, h_hi, gin_lo, gin_hi, gout_lo, gout_hi, a_in, a_out,
            indeg, outdeg,
            wef[:d], wef[d:2 * d], wer[:d], wer[d:2 * d], wfa, wra,
            wih[:, :d], wih[:, d:2 * d], wih[:, 2 * d:],
            whh[:, :d], whh[:, d:2 * d], whh[:, 2 * d:],
            b_ih[i][None, :d], b_ih[i][None, d:2 * d], b_ih[i][None, 2 * d:],
            b_hh[i][None, :d], b_hh[i][None, d:2 * d], b_hh[i][None, 2 * d:])
        if i + 1 < rounds:
            h_lo, h_hi = round_call(*args)
        else:
            h_lo, h_hi, h_g = last_call(
                *args, Wfm, bfm[None, :], Wgm, bgm[None, :])
    return h_g


# speedup vs baseline: 1.0698x; 1.0698x over previous
import functools

import jax
import jax.numpy as jnp
from jax import lax
from jax.experimental import pallas as pl
from jax.experimental.pallas import tpu as pltpu
from jax.experimental.pallas import tpu_sc as plsc

NC = 2
NS = 16
L = 128
IB = 8
S = 4


def _sc_mesh():
    return plsc.VectorSubcoreMesh(
        core_axis_name="c", subcore_axis_name="s", num_cores=NC,
        num_subcores=NS)


def _zero_zbuf(zbuf, zb_rows, width):
    z16 = jnp.zeros((16,), jnp.float32)

    def zrow(i, _):
        for w in range(width // 16):
            zbuf[i, pl.ds(w * 16, 16)] = z16
        return 0

    lax.fori_loop(0, zb_rows, zrow, 0)


def _zero_acc_slice(acc, zbuf, tid, npt, zb):
    def body(z, _):
        pltpu.sync_copy(zbuf, acc.at[pl.ds(tid * npt + z * zb, zb)])
        return 0

    lax.fori_loop(0, npt // zb, body, 0)


def _dump_acc_slice(acc, out_hbm, tid, npt, zb):
    def body(z, _):
        r0 = tid * npt + z * zb
        pltpu.sync_copy(acc.at[pl.ds(r0, zb)], out_hbm.at[pl.ds(r0, zb)])
        return 0

    lax.fori_loop(0, npt // zb, body, 0)


def _pipe_block(fire_fetch, sblk, rows_list, acc, gsems, ssems):
    prev = [None] * S
    for p in range(IB // S):
        j0 = S * p
        g = []
        for j in range(S):
            if prev[j] is not None:
                prev[j].wait()
            g.append(fire_fetch(j0 + j, rows_list[j], gsems[j]))
        for j in range(S):
            g[j].wait()
            prev[j] = pltpu.async_copy(rows_list[j],
                                       acc.at[sblk.at[j0 + j]],
                                       ssems[j], add=True)
    for d in prev:
        d.wait()


def _make_gsum_kernel(n_out, h, e_pad, npt, zb, kpt):
    kb = e_pad // L
    assert kb == NS * kpt and kpt % IB == 0
    nblk = kpt // IB

    out4 = tuple(jax.ShapeDtypeStruct((n_out, h), jnp.float32)
                 for _ in range(4))
    scratch = (
        [pltpu.VMEM((IB, L), jnp.int32)] * 2
        + [pltpu.VMEM((L, h), jnp.float32)] * S
        + [pltpu.VMEM((zb, h), jnp.float32)]
        + [pltpu.VMEM_SHARED((n_out, h), jnp.float32)]
        + [pltpu.SemaphoreType.DMA] * (2 * S))

    @functools.partial(pl.kernel, out_type=out4, mesh=_sc_mesh(),
                       scratch_types=scratch,
                       compiler_params=pltpu.CompilerParams(
                           use_tc_tiling_on_sc=False))
    def gsum(h_lo, h_hi, gsrc2, gdst2, ssrc2, sdst2,
             gin_lo, gin_hi, gout_lo, gout_hi,
             gblk, sblk, *rest):
        rows_list = list(rest[:S])
        zbuf = rest[S]
        acc = rest[S + 1]
        gsems = list(rest[S + 2:2 * S + 2])
        ssems = list(rest[2 * S + 2:])
        cid = lax.axis_index("c")
        tid = lax.axis_index("s")

        _zero_zbuf(zbuf, zb, h)

        def run_pass(gidx2, sidx2, table):
            def blk(b, _):
                b0 = tid * kpt + b * IB
                pltpu.sync_copy(gidx2.at[pl.ds(b0, IB)], gblk)
                pltpu.sync_copy(sidx2.at[pl.ds(b0, IB)], sblk)
                _pipe_block(
                    lambda j, dst, sem: pltpu.async_copy(
                        table.at[gblk.at[j]], dst, sem),
                    sblk, rows_list, acc, gsems, ssems)
                return 0

            lax.fori_loop(0, nblk, blk, 0)

        def flow(table, o_in, o_out):
            _zero_acc_slice(acc, zbuf, tid, npt, zb)
            plsc.subcore_barrier()
            run_pass(gsrc2, sdst2, table)
            plsc.subcore_barrier()
            _dump_acc_slice(acc, o_in, tid, npt, zb)
            _zero_acc_slice(acc, zbuf, tid, npt, zb)
            plsc.subcore_barrier()
            run_pass(gdst2, ssrc2, table)
            plsc.subcore_barrier()
            _dump_acc_slice(acc, o_out, tid, npt, zb)

        @pl.when(cid == 0)
        def _():
            flow(h_lo, gin_lo, gout_lo)

        @pl.when(cid == 1)
        def _():
            flow(h_hi, gin_hi, gout_hi)

    return gsum


def _make_attr_kernel(n_out, e_pad, npt, zb, kpt):
    w = 16
    nblk = kpt // IB

    out2 = (jax.ShapeDtypeStruct((n_out, w), jnp.float32),
            jax.ShapeDtypeStruct((n_out, w), jnp.float32))
    scratch = (
        [pltpu.VMEM((IB, L), jnp.int32)]
        + [pltpu.VMEM((L, w), jnp.float32)] * S
        + [pltpu.VMEM((zb, w), jnp.float32)]
        + [pltpu.VMEM_SHARED((n_out, w), jnp.float32)]
        + [pltpu.SemaphoreType.DMA] * (2 * S))

    @functools.partial(pl.kernel, out_type=out2, mesh=_sc_mesh(),
                       scratch_types=scratch,
                       compiler_params=pltpu.CompilerParams(
                           use_tc_tiling_on_sc=False))
    def attr_sum(attr16, sdst2, ssrc2, a_in, a_out, sblk, *rest):
        rows_list = list(rest[:S])
        zbuf = rest[S]
        acc = rest[S + 1]
        gsems = list(rest[S + 2:2 * S + 2])
        ssems = list(rest[2 * S + 2:])
        cid = lax.axis_index("c")
        tid = lax.axis_index("s")

        _zero_zbuf(zbuf, zb, w)

        def flow(sidx2, out_hbm):
            _zero_acc_slice(acc, zbuf, tid, npt, zb)
            plsc.subcore_barrier()

            def blk(b, _):
                b0 = tid * kpt + b * IB
                pltpu.sync_copy(sidx2.at[pl.ds(b0, IB)], sblk)
                _pipe_block(
                    lambda j, dst, sem: pltpu.async_copy(
                        attr16.at[pl.ds((b0 + j) * L, L)], dst, sem),
                    sblk, rows_list, acc, gsems, ssems)
                return 0

            lax.fori_loop(0, nblk, blk, 0)
            plsc.subcore_barrier()
            _dump_acc_slice(acc, out_hbm, tid, npt, zb)

        @pl.when(cid == 0)
        def _():
            flow(sdst2, a_in)

        @pl.when(cid == 1)
        def _():
            flow(ssrc2, a_out)

    return attr_sum


def _round_body(lo, hi, ginlo, ginhi, goutlo, gouthi, ain, aout, indeg,
                outdeg, wefs, wefd, wers, werd, wfa, wra,
                wihr, wihz, wihn, whhr, whhz, whhn,
                bir, biz, bin_, bhr, bhz, bhn, olo, ohi):
    h = jnp.concatenate([lo[...], hi[...]], axis=1)
    gin = jnp.concatenate([ginlo[...], ginhi[...]], axis=1)
    gout = jnp.concatenate([goutlo[...], gouthi[...]], axis=1)
    mm = lambda a, b: jnp.dot(a, b, preferred_element_type=jnp.float32)
    agg = (mm(gin, wefs[...]) + mm(gout, werd[...])
           + indeg[...] * mm(h, wefd[...]) + outdeg[...] * mm(h, wers[...])
           + mm(ain[...], wfa[...]) + mm(aout[...], wra[...]))
    r = jax.nn.sigmoid(mm(agg, wihr[...]) + bir[...]
                       + mm(h, whhr[...]) + bhr[...])
    z = jax.nn.sigmoid(mm(agg, wihz[...]) + biz[...]
                       + mm(h, whhz[...]) + bhz[...])
    nn = jnp.tanh(mm(agg, wihn[...]) + bin_[...]
                  + r * (mm(h, whhn[...]) + bhn[...]))
    hn = (1.0 - z) * nn + z * h
    half = hn.shape[1] // 2
    olo[...] = hn[:, :half]
    ohi[...] = hn[:, half:]


def _round_agg_body(lo, hi, ginlo, ginhi, goutlo, gouthi, ain, aout, indeg,
                    outdeg, wefs, wefd, wers, werd, wfa, wra,
                    wihr, wihz, wihn, whhr, whhz, whhn,
                    bir, biz, bin_, bhr, bhz, bhn,
                    wfm, bfm, wgm, bgm, olo, ohi, oagg):
    _round_body(lo, hi, ginlo, ginhi, goutlo, gouthi, ain, aout, indeg,
                outdeg, wefs, wefd, wers, werd, wfa, wra,
                wihr, wihz, wihn, whhr, whhz, whhn,
                bir, biz, bin_, bhr, bhz, bhn, olo, ohi)
    hn = jnp.concatenate([olo[...], ohi[...]], axis=1)
    mm = lambda a, b: jnp.dot(a, b, preferred_element_type=jnp.float32)
    f = mm(hn, wfm[...]) + bfm[...]
    g = jax.nn.sigmoid(mm(hn, wgm[...]) + bgm[...])
    p = jnp.sum(f * g, axis=0, keepdims=True)

    @pl.when(pl.program_id(0) == 0)
    def _():
        oagg[...] = jnp.zeros_like(oagg)

    oagg[...] += p


def kernel(x, edge_index, edge_attr, Wef, bef, Wer, ber, W_ih, W_hh,
           b_ih, b_hh, Wfm, bfm, Wgm, bgm):
    n, d = x.shape
    h = d // 2
    e = edge_index.shape[1]
    es = edge_attr.shape[1]
    rounds = Wef.shape[0]
    hg = Wfm.shape[1]

    kpt = IB * -(-e // (NS * IB * L))
    e_pad = NS * kpt * L
    n_out = -(-n // (NS * 8)) * (NS * 8)
    npt = n_out // NS
    zb = next(c for c in (512, 256, 184, 136, 128, 64, 32, 16, 8)
              if npt % c == 0)
    trash = n_out - 8

    src = edge_index[0]
    dst = edge_index[1]
    pad_g = jnp.zeros((e_pad - e,), jnp.int32)
    pad_s = jnp.full((e_pad - e,), trash, jnp.int32)
    gsrc2 = jnp.concatenate([src, pad_g]).reshape(-1, L)
    gdst2 = jnp.concatenate([dst, pad_g]).reshape(-1, L)
    ssrc2 = jnp.concatenate([src, pad_s]).reshape(-1, L)
    sdst2 = jnp.concatenate([dst, pad_s]).reshape(-1, L)
    attr16 = jnp.pad(
        jnp.concatenate([edge_attr, jnp.ones((e, 1), jnp.float32)], axis=1),
        ((0, e_pad - e), (0, 16 - es - 1)))

    attr_kernel = _make_attr_kernel(n_out, e_pad, npt, zb, kpt)
    a_in, a_out = attr_kernel(attr16, sdst2, ssrc2)
    indeg = a_in[:, es:es + 1]
    outdeg = a_out[:, es:es + 1]

    gsum = _make_gsum_kernel(n_out, h, e_pad, npt, zb, kpt)

    bn = 2000
    grid = (n // bn,)
    row_spec = lambda w: pl.BlockSpec((bn, w), lambda i: (i, 0))
    full_spec = lambda a, b: pl.BlockSpec((a, b), lambda i: (0, 0))

    wspecs = ([full_spec(d, 2 * d)] * 2 + [full_spec(d, 2 * d)] * 2
              + [full_spec(16, 2 * d)] * 2
              + [full_spec(2 * d, d)] * 3 + [full_spec(d, d)] * 3
              + [full_spec(1, d)] * 6)
    round_call = pl.pallas_call(
        _round_body,
        grid=grid,
        in_specs=[row_spec(h)] * 6 + [row_spec(16)] * 2 + [row_spec(1)] * 2
                 + wspecs,
        out_specs=[row_spec(h), row_spec(h)],
        out_shape=[jax.ShapeDtypeStruct((n, h), jnp.float32)] * 2,
    )

    last_call = pl.pallas_call(
        _round_agg_body,
        grid=grid,
        in_specs=[row_spec(h)] * 6 + [row_spec(16)] * 2 + [row_spec(1)] * 2
                 + wspecs + [full_spec(d, hg), full_spec(1, hg),
                             full_spec(d, hg), full_spec(1, hg)],
        out_specs=[row_spec(h), row_spec(h),
                   pl.BlockSpec((1, hg), lambda i: (0, 0))],
        out_shape=[jax.ShapeDtypeStruct((n, h), jnp.float32)] * 2
                  + [jax.ShapeDtypeStruct((1, hg), jnp.float32)],
    )

    h_lo = x[:, :h]
    h_hi = x[:, h:]
    h_g = None
    for i in range(rounds):
        gin_lo, gin_hi, gout_lo, gout_hi = gsum(
            h_lo, h_hi, gsrc2, gdst2, ssrc2, sdst2)
        wef, wer = Wef[i], Wer[i]
        wfa = jnp.concatenate(
            [wef[2 * d:], bef[i][None, :],
             jnp.zeros((16 - es - 1, 2 * d), jnp.float32)], axis=0)
        wra = jnp.concatenate(
            [wer[2 * d:], ber[i][None, :],
             jnp.zeros((16 - es - 1, 2 * d), jnp.float32)], axis=0)
        wih, whh = W_ih[i], W_hh[i]
        args = (
            h_lo, h_hi, gin_lo, gin_hi, gout_lo, gout_hi, a_in, a_out,
            indeg, outdeg,
            wef[:d], wef[d:2 * d], wer[:d], wer[d:2 * d], wfa, wra,
            wih[:, :d], wih[:, d:2 * d], wih[:, 2 * d:],
            whh[:, :d], whh[:, d:2 * d], whh[:, 2 * d:],
            b_ih[i][None, :d], b_ih[i][None, d:2 * d], b_ih[i][None, 2 * d:],
            b_hh[i][None, :d], b_hh[i][None, d:2 * d], b_hh[i][None, 2 * d:])
        if i + 1 < rounds:
            h_lo, h_hi = round_call(*args)
        else:
            h_lo, h_hi, h_g = last_call(
                *args, Wfm, bfm[None, :], Wgm, bgm[None, :])
    return h_g
